# baseline (speedup 1.0000x reference)
import jax
import jax.numpy as jnp
from jax.experimental import pallas as pl

BATCH = 8
X_DIM = 200
Y_DIM = 200
Z_DIM = 80
VOXEL = 50.0

N_PIX = BATCH * X_DIM * Y_DIM
ROWS = 2000
GRID = N_PIX // ROWS


def _volume_body(lwp_ref, occ_ref, cbh_ref, dh_ref, out_ref):
    lwp = jnp.maximum(lwp_ref[...], 0.0)
    occ = occ_ref[...]
    cbh = jnp.maximum(cbh_ref[...] * 1000.0, 0.0)
    dh = jnp.maximum(dh_ref[...] * 1000.0, 0.0)

    cbh = jnp.round(cbh / VOXEL) * VOXEL
    dh = jnp.round(dh / VOXEL) * VOXEL

    mask = occ > 0.0
    valid = mask & (cbh < Z_DIM * VOXEL)

    cbh_idx = cbh * (1.0 / VOXEL)
    top_idx = jnp.minimum((cbh + dh) * (1.0 / VOXEL), float(Z_DIM - 1))

    dh_safe = jnp.where(dh > 0.0, dh, 1.0)
    inv_dh = 1.0 / dh_safe
    coeff = 2.0 * lwp * inv_dh * inv_dh

    zi = jax.lax.broadcasted_iota(jnp.int32, (ROWS, Z_DIM), 1)
    zf = zi.astype(jnp.float32)
    in_band = (zf >= cbh_idx) & (zf < top_idx) & valid
    h = zf * VOXEL + (VOXEL / 2.0) - cbh
    out_ref[...] = jnp.where(in_band, coeff * h, 0.0)


def kernel(lwp, occupancy_logits, cloud_base_heights, delta_heights_pred):
    flat = lambda a: a.reshape(N_PIX, 1)
    out = pl.pallas_call(
        _volume_body,
        grid=(GRID,),
        in_specs=[pl.BlockSpec((ROWS, 1), lambda i: (i, 0))] * 4,
        out_specs=pl.BlockSpec((ROWS, Z_DIM), lambda i: (i, 0)),
        out_shape=jax.ShapeDtypeStruct((N_PIX, Z_DIM), jnp.float32),
    )(flat(lwp), flat(occupancy_logits), flat(cloud_base_heights),
      flat(delta_heights_pred))
    return out.reshape(BATCH, 1, X_DIM, Y_DIM, Z_DIM)


# speedup vs baseline: 2.4630x; 2.4630x over previous
import jax
import jax.numpy as jnp
from jax.experimental import pallas as pl

BATCH = 8
X_DIM = 200
Y_DIM = 200
Z_DIM = 80
VOXEL = 50.0

BX = 8
GRID_X = X_DIM // BX


def _volume_body(lwp_ref, occ_ref, cbh_ref, dh_ref, out_ref):
    lwp = jnp.maximum(lwp_ref[...], 0.0)
    occ = occ_ref[...]
    cbh = jnp.maximum(cbh_ref[...] * 1000.0, 0.0)
    dh = jnp.maximum(dh_ref[...] * 1000.0, 0.0)

    cbh_idx = jnp.round(cbh / VOXEL)
    dh = jnp.round(dh / VOXEL) * VOXEL

    mask = occ > 0.0
    valid = mask & (cbh_idx < float(Z_DIM))

    top = jnp.minimum(cbh_idx + dh * (1.0 / VOXEL), float(Z_DIM - 1))
    top = jnp.where(valid, top, -1.0)

    dh_safe = jnp.where(dh > 0.0, dh, 1.0)
    inv_dh = 1.0 / dh_safe
    coeff = 2.0 * lwp * inv_dh * inv_dh

    a3 = cbh_idx[:, :, None]
    t3 = top[:, :, None]
    c3 = coeff[:, :, None]

    zi = jax.lax.broadcasted_iota(jnp.int32, (BX, Y_DIM, Z_DIM), 2)
    zf = zi.astype(jnp.float32)
    in_band = (zf >= a3) & (zf < t3)
    h = (zf - a3) * VOXEL + (VOXEL / 2.0)
    out_ref[0, 0] = jnp.where(in_band, c3 * h, 0.0)


def kernel(lwp, occupancy_logits, cloud_base_heights, delta_heights_pred):
    flat = lambda a: a.reshape(BATCH * X_DIM, Y_DIM)
    in_spec = pl.BlockSpec((BX, Y_DIM), lambda b, xb: (b * GRID_X + xb, 0))
    out = pl.pallas_call(
        _volume_body,
        grid=(BATCH, GRID_X),
        in_specs=[in_spec] * 4,
        out_specs=pl.BlockSpec((1, 1, BX, Y_DIM, Z_DIM),
                               lambda b, xb: (b, 0, xb, 0, 0)),
        out_shape=jax.ShapeDtypeStruct(
            (BATCH, 1, X_DIM, Y_DIM, Z_DIM), jnp.float32),
    )(flat(lwp), flat(occupancy_logits), flat(cloud_base_heights),
      flat(delta_heights_pred))
    return out
